# baseline (speedup 1.0000x reference)
import functools

import jax
import jax.numpy as jnp
from jax import lax
from jax.experimental import pallas as pl
from jax.experimental.pallas import tpu as pltpu
from jax.experimental.pallas import tpu_sc as plsc

N = 10000
NHE = 10000
E = 160000
FIN = 256
NHID = 512
NGRAPH = 16
DPIN = 16
FC = 128

ACC_ROWS = 10240
DUMMY = 10000
P = 64

KB = 128
NBLK = 80
NBLK_C = 40
EPAD = 163840


def _leaky(x):
    return jnp.where(x >= 0, x, 0.1 * x)


def _sc_mesh():
    return plsc.VectorSubcoreMesh(core_axis_name="c", subcore_axis_name="s")


def _zero_acc(sid, sbuf, acc):
    npiece = ACC_ROWS // (16 * P)
    for z in range(npiece):
        pltpu.sync_copy(sbuf, acc.at[pl.ds((sid * npiece + z) * P, P)])


def _writeback(sid, sbuf, acc, dst):
    def piece(base, rows):
        pltpu.sync_copy(acc.at[pl.ds(base, rows)], sbuf.at[pl.ds(0, rows)])
        pltpu.sync_copy(sbuf.at[pl.ds(0, rows)], dst.at[pl.ds(base, rows)])

    for w in range(9):
        piece((sid + 16 * w) * P, P)

    @pl.when(sid < 12)
    def _():
        piece(144 * P + sid * P, P)

    @pl.when(sid == 12)
    def _():
        piece(156 * P, 16)


def _spmm(nchunk, src3, dst3, table, zeros_hbm, split_last=False):
    nfull = nchunk // 2
    nout = nchunk + 1 if split_last else nchunk
    assert not (split_last and nchunk % 2 == 0)
    G = 8
    NGRP = NBLK // G

    @functools.partial(
        pl.kernel,
        mesh=_sc_mesh(),
        out_type=jax.ShapeDtypeStruct((nout, N, FC), jnp.float32),
        scratch_types=[
            pltpu.VMEM((G, KB), jnp.int32),
            pltpu.VMEM((G, KB), jnp.int32),
            pltpu.VMEM((KB, FC), jnp.float32),
            pltpu.VMEM((KB, FC), jnp.float32),
            pltpu.VMEM((P, FC), jnp.float32),
            pltpu.VMEM_SHARED((ACC_ROWS, FC), jnp.float32),
            pltpu.SemaphoreType.DMA,
            pltpu.SemaphoreType.DMA,
        ],
    )
    def k(src_h, dst_h, tab_h, z_h, out_h,
          sidx, didx, rbuf0, rbuf1, sbuf, acc, sem0, sem1):
        cid = lax.axis_index("c")
        sid = lax.axis_index("s")
        src_t = src_h.at[sid]
        dst_t = dst_h.at[sid]
        bufs = (rbuf0, rbuf1)
        sems = (sem0, sem1)

        def chunk(c_tab, c_out, g_lo, g_hi):
            pltpu.sync_copy(z_h, sbuf)
            _zero_acc(sid, sbuf, acc)
            plsc.subcore_barrier()
            tab_c = tab_h.at[c_tab]

            def group(g, carry):
                pltpu.sync_copy(src_t.at[pl.ds(g * G, G)], sidx)
                pltpu.sync_copy(dst_t.at[pl.ds(g * G, G)], didx)
                handles = [None, None]
                handles[0] = pltpu.async_copy(
                    tab_c.at[sidx.at[0]], bufs[0], sems[0])
                for r in range(G):
                    if r + 1 < G:
                        handles[(r + 1) % 2] = pltpu.async_copy(
                            tab_c.at[sidx.at[r + 1]],
                            bufs[(r + 1) % 2], sems[(r + 1) % 2])
                    handles[r % 2].wait()
                    pltpu.sync_copy(bufs[r % 2], acc.at[didx.at[r]], add=True)
                return carry

            lax.fori_loop(g_lo, g_hi, group, 0)
            plsc.subcore_barrier()
            _writeback(sid, sbuf, acc, out_h.at[c_out])

        for l in range(nfull):
            if l > 0:
                plsc.subcore_barrier()
            chunk(cid + 2 * l, cid + 2 * l, 0, NGRP)
        if split_last:
            if nfull > 0:
                plsc.subcore_barrier()
            half = NGRP // 2
            chunk(nchunk - 1, nchunk - 1 + cid,
                  cid * half, cid * half + half)
        elif nchunk % 2 == 1:
            if nfull > 0:
                plsc.subcore_barrier()

            @pl.when(cid == 0)
            def _():
                chunk(nchunk - 1, nchunk - 1, 0, NGRP)

    return k(src3, dst3, table, zeros_hbm)


def _counts_a(col3, pinb_hbm, zeros_hbm):
    @functools.partial(
        pl.kernel,
        mesh=_sc_mesh(),
        out_type=jax.ShapeDtypeStruct((2, NHE, 128), jnp.float32),
        scratch_types=[
            pltpu.VMEM((NBLK_C, KB), jnp.int32),
            pltpu.VMEM((KB, 128), jnp.float32),
            pltpu.VMEM((P, 128), jnp.float32),
            pltpu.VMEM_SHARED((ACC_ROWS, 128), jnp.float32),
            pltpu.SemaphoreType.DMA,
        ],
    )
    def k(col_h, pin_h, z_h, qb_out, colv, pbuf, sbuf, acc, sem):
        cid = lax.axis_index("c")
        sid = lax.axis_index("s")
        j = cid * 16 + sid
        pltpu.sync_copy(col_h.at[j], colv)
        pltpu.sync_copy(z_h, sbuf)
        _zero_acc(sid, sbuf, acc)
        plsc.subcore_barrier()
        ebase = j * (NBLK_C * KB)

        def body(b, carry):
            pltpu.async_copy(pin_h.at[pl.ds(ebase + b * KB, KB)], pbuf,
                             sem).wait()
            pltpu.sync_copy(pbuf, acc.at[colv.at[b]], add=True)
            return carry

        lax.fori_loop(0, NBLK_C, body, 0)
        plsc.subcore_barrier()
        _writeback(sid, sbuf, acc, qb_out.at[cid])

    return k(col3, pinb_hbm, zeros_hbm)


BN = 2000
BNM = 2000


def _prep(x, fake_pos, macro1x):
    def body(x_ref, fp_ref, mac_ref, o_ref):
        xb = x_ref[...]
        nid = (pl.program_id(0) * BN
               + lax.broadcasted_iota(jnp.int32, (BN, 1), 0))
        hit = (nid == mac_ref[...]).astype(jnp.float32)
        ism = jnp.minimum(jnp.sum(hit, axis=1, keepdims=True), 1.0)
        z = jnp.zeros((BN, 125), jnp.float32)
        o_ref[0] = xb[:, :128]
        o_ref[1] = xb[:, 128:]
        o_ref[2] = jnp.concatenate([fp_ref[...], ism, z], axis=1)

    return pl.pallas_call(
        body,
        grid=(N // BN,),
        in_specs=[
            pl.BlockSpec((BN, FIN), lambda r: (r, 0)),
            pl.BlockSpec((BN, 2), lambda r: (r, 0)),
            pl.BlockSpec((1, 512), lambda r: (0, 0)),
        ],
        out_specs=pl.BlockSpec((3, BN, 128), lambda r: (0, r, 0)),
        out_shape=jax.ShapeDtypeStruct((3, N, 128), jnp.float32),
    )(x, fake_pos, macro1x)


def _scale1(u, qb):
    def body(u_ref, qb_ref, o_ref):
        bsum = qb_ref[0, :, 16:17] + qb_ref[1, :, 16:17]
        binv = jnp.where(bsum > 0, 1.0 / bsum, 0.0)
        q = qb_ref[0, :, :16] + qb_ref[1, :, :16]
        c2 = u_ref[2] + u_ref[3]
        o_ref[0] = u_ref[0] * binv
        o_ref[1] = u_ref[1] * binv
        o_ref[2] = jnp.concatenate(
            [c2[:, :3] * binv, q * binv, jnp.ones((BN, 1), jnp.float32),
             jnp.zeros((BN, 108), jnp.float32)], axis=1)

    return pl.pallas_call(
        body,
        grid=(NHE // BN,),
        in_specs=[
            pl.BlockSpec((4, BN, 128), lambda r: (0, r, 0)),
            pl.BlockSpec((2, BN, 128), lambda r: (0, r, 0)),
        ],
        out_specs=pl.BlockSpec((3, BN, 128), lambda r: (0, r, 0)),
        out_shape=jax.ShapeDtypeStruct((3, NHE, 128), jnp.float32),
    )(u, qb)


def _scale(u, qb):
    def body(u_ref, qb_ref, o_ref):
        bsum = qb_ref[0, :, 16:17] + qb_ref[1, :, 16:17]
        binv = jnp.where(bsum > 0, 1.0 / bsum, 0.0)
        o_ref[...] = u_ref[...] * binv[None]

    return pl.pallas_call(
        body,
        grid=(NHE // BN,),
        in_specs=[
            pl.BlockSpec((4, BN, 128), lambda r: (0, r, 0)),
            pl.BlockSpec((2, BN, 128), lambda r: (0, r, 0)),
        ],
        out_specs=pl.BlockSpec((4, BN, 128), lambda r: (0, r, 0)),
        out_shape=jax.ShapeDtypeStruct((4, NHE, 128), jnp.float32),
    )(u, qb)


def _layer_matmul(v, wc, b2d, dm, nc):
    def body(v_ref, w_ref, b_ref, dm_ref, o_ref):
        acc = jnp.zeros((BNM, 128), jnp.float32)
        for kk in range(nc):
            acc = acc + jnp.dot(v_ref[kk], w_ref[kk],
                                preferred_element_type=jnp.float32)
        dsum = dm_ref[0, :, 19:20] + dm_ref[1, :, 19:20]
        dinv = jnp.where(dsum > 0, 1.0 / dsum, 0.0)
        o_ref[0] = _leaky(acc * dinv + b_ref[0, 0][None])

    return pl.pallas_call(
        body,
        grid=(N // BNM, 4),
        in_specs=[
            pl.BlockSpec((nc, BNM, 128), lambda r, c: (0, r, 0)),
            pl.BlockSpec((nc, 128, 128), lambda r, c: (0, 0, c)),
            pl.BlockSpec((1, 1, 128), lambda r, c: (c, 0, 0)),
            pl.BlockSpec((2, BNM, 128), lambda r, c: (0, r, 0)),
        ],
        out_specs=pl.BlockSpec((1, BNM, 128), lambda r, c: (c, r, 0)),
        out_shape=jax.ShapeDtypeStruct((4, N, 128), jnp.float32),
    )(v, wc, b2d, dm)


def _pool(hc, batch2d, macro1x):
    RN = N // BN

    def body(h_ref, b_ref, mac_ref, o_ref, s_ref, sm_ref, c_ref, cm_ref):
        r = pl.program_id(0)

        @pl.when(r == 0)
        def _():
            s_ref[...] = jnp.zeros_like(s_ref)
            sm_ref[...] = jnp.zeros_like(sm_ref)
            c_ref[...] = jnp.zeros_like(c_ref)
            cm_ref[...] = jnp.zeros_like(cm_ref)

        ids = b_ref[...]
        oh = (ids == lax.broadcasted_iota(jnp.int32, (BN, NGRAPH), 1))
        oh = oh.astype(jnp.float32)
        nid = (r * BN + lax.broadcasted_iota(jnp.int32, (BN, 1), 0))
        mcnt = jnp.sum((nid == mac_ref[...]).astype(jnp.float32),
                       axis=1, keepdims=True)
        h = jnp.concatenate(
            [h_ref[0], h_ref[1], h_ref[2], h_ref[3]], axis=1)
        dn = (((0,), (0,)), ((), ()))
        s_ref[...] += lax.dot_general(oh, h, dn,
                                      preferred_element_type=jnp.float32)
        sm_ref[...] += lax.dot_general(oh, mcnt * h, dn,
                                       preferred_element_type=jnp.float32)
        ones = jnp.ones((BN, 1), jnp.float32)
        c_ref[...] += lax.dot_general(oh, ones, dn,
                                      preferred_element_type=jnp.float32)
        cm_ref[...] += lax.dot_general(oh, mcnt, dn,
                                       preferred_element_type=jnp.float32)

        @pl.when(r == RN - 1)
        def _():
            sm = sm_ref[...] / jnp.maximum(cm_ref[...], 1.0)
            s = s_ref[...] / jnp.maximum(c_ref[...], 1.0)
            o_ref[...] = jnp.concatenate([sm, s], axis=1)

    return pl.pallas_call(
        body,
        grid=(RN,),
        in_specs=[
            pl.BlockSpec((4, BN, 128), lambda r: (0, r, 0)),
            pl.BlockSpec((BN, 1), lambda r: (r, 0)),
            pl.BlockSpec((1, 512), lambda r: (0, 0)),
        ],
        out_specs=pl.BlockSpec((NGRAPH, 2 * NHID), lambda r: (0, 0)),
        out_shape=jax.ShapeDtypeStruct((NGRAPH, 2 * NHID), jnp.float32),
        scratch_shapes=[
            pltpu.VMEM((NGRAPH, NHID), jnp.float32),
            pltpu.VMEM((NGRAPH, NHID), jnp.float32),
            pltpu.VMEM((NGRAPH, 1), jnp.float32),
            pltpu.VMEM((NGRAPH, 1), jnp.float32),
        ],
    )(hc, batch2d, macro1x)


def _mlp(x1, x2, x3, mw1, mb1, mw2, mb2, mw3, mb3):
    def body(x1_r, x2_r, x3_r, w1_r, b1_r, w2_r, b2_r, w3_r, b3_r, o_ref):
        g = x1_r[...] + x2_r[...] + x3_r[...]
        g = _leaky(jnp.dot(g, w1_r[...],
                           preferred_element_type=jnp.float32) + b1_r[...])
        g = _leaky(jnp.dot(g, w2_r[...],
                           preferred_element_type=jnp.float32) + b2_r[...])
        o_ref[...] = jnp.dot(g, w3_r[...],
                             preferred_element_type=jnp.float32) + b3_r[...]

    return pl.pallas_call(
        body,
        out_shape=jax.ShapeDtypeStruct((NGRAPH, 4), jnp.float32),
    )(x1, x2, x3, mw1, mb1.reshape(1, -1), mw2, mb2.reshape(1, -1),
      mw3, mb3.reshape(1, -1))


def kernel(x, edge_index, pin_feature, batch, fake_pos, macro_index,
           W1, pinW1, b1, W2, b2, W3, b3, mw1, mb1, mw2, mb2, mw3, mb3):
    row = edge_index[0].astype(jnp.int32)
    col = edge_index[1].astype(jnp.int32)
    pad = EPAD - E

    def pad_idx(a, fill):
        return jnp.concatenate([a, jnp.full((pad,), fill, jnp.int32)])

    row_s = pad_idx(row, 0).reshape(16, NBLK, KB)
    row_d = pad_idx(row, DUMMY).reshape(16, NBLK, KB)
    col_s = pad_idx(col, 0).reshape(16, NBLK, KB)
    col_d = pad_idx(col, DUMMY).reshape(16, NBLK, KB)
    col_c = pad_idx(col, DUMMY).reshape(32, NBLK_C, KB)
    pinb = jnp.concatenate(
        [pin_feature, jnp.ones((E, 1), jnp.float32),
         jnp.zeros((E, 111), jnp.float32)], axis=1)
    pinb = jnp.concatenate([pinb, jnp.zeros((pad, 128), jnp.float32)])
    macro1x = macro_index.astype(jnp.int32).reshape(1, 512)
    z128 = jnp.zeros((P, 128), jnp.float32)

    qb = _counts_a(col_c, pinb, z128)

    h0c = _prep(x, fake_pos, macro1x)
    u1 = _spmm(3, row_s, col_d, h0c, z128, split_last=True)
    u1s = _scale1(u1, qb)
    v1 = _spmm(3, col_s, row_d, u1s, z128, split_last=True)
    w_c2 = jnp.concatenate([
        W1[256:259], pinW1, jnp.zeros((109, NHID), jnp.float32)])
    bigW1 = jnp.concatenate([W1[:256], w_c2, w_c2]).reshape(4, 128, NHID)
    dpar = v1[2:4]
    h1 = _layer_matmul(v1, bigW1, b1.reshape(4, 1, 128), dpar, 4)

    w2c = W2.reshape(4, 128, NHID)
    u2 = _spmm(4, row_s, col_d, h1, z128)
    v2 = _spmm(4, col_s, row_d, _scale(u2, qb), z128)
    h2 = _layer_matmul(v2, w2c, b2.reshape(4, 1, 128), dpar, 4)

    w3c = W3.reshape(4, 128, NHID)
    u3 = _spmm(4, row_s, col_d, h2, z128)
    v3 = _spmm(4, col_s, row_d, _scale(u3, qb), z128)
    h3 = _layer_matmul(v3, w3c, b3.reshape(4, 1, 128), dpar, 4)

    batch2d = batch.astype(jnp.int32).reshape(N, 1)
    x1 = _pool(h1, batch2d, macro1x)
    x2 = _pool(h2, batch2d, macro1x)
    x3 = _pool(h3, batch2d, macro1x)

    return _mlp(x1, x2, x3, mw1, mb1, mw2, mb2, mw3, mb3)


# speedup vs baseline: 1.0127x; 1.0127x over previous
import functools

import jax
import jax.numpy as jnp
from jax import lax
from jax.experimental import pallas as pl
from jax.experimental.pallas import tpu as pltpu
from jax.experimental.pallas import tpu_sc as plsc

N = 10000
NHE = 10000
E = 160000
FIN = 256
NHID = 512
NGRAPH = 16
DPIN = 16
FC = 128

ACC_ROWS = 10240
DUMMY = 10000
P = 64

KB = 128
NBLK = 80
NBLK_C = 40
EPAD = 163840


def _leaky(x):
    return jnp.where(x >= 0, x, 0.1 * x)


def _sc_mesh():
    return plsc.VectorSubcoreMesh(core_axis_name="c", subcore_axis_name="s")


def _zero_acc(sid, sbuf, acc):
    npiece = ACC_ROWS // (16 * P)
    for z in range(npiece):
        pltpu.sync_copy(sbuf, acc.at[pl.ds((sid * npiece + z) * P, P)])


def _writeback(sid, sbuf, acc, dst):
    def piece(base, rows):
        pltpu.sync_copy(acc.at[pl.ds(base, rows)], sbuf.at[pl.ds(0, rows)])
        pltpu.sync_copy(sbuf.at[pl.ds(0, rows)], dst.at[pl.ds(base, rows)])

    for w in range(9):
        piece((sid + 16 * w) * P, P)

    @pl.when(sid < 12)
    def _():
        piece(144 * P + sid * P, P)

    @pl.when(sid == 12)
    def _():
        piece(156 * P, 16)


def _spmm(nchunk, src3, dst3, table, zeros_hbm, split_last=False):
    nfull = nchunk // 2
    nout = nchunk + 1 if split_last else nchunk
    assert not (split_last and nchunk % 2 == 0)
    G = 8
    NGRP = NBLK // G

    @functools.partial(
        pl.kernel,
        mesh=_sc_mesh(),
        out_type=jax.ShapeDtypeStruct((nout, N, FC), jnp.float32),
        scratch_types=[
            pltpu.VMEM((G, KB), jnp.int32),
            pltpu.VMEM((G, KB), jnp.int32),
            pltpu.VMEM((KB, FC), jnp.float32),
            pltpu.VMEM((KB, FC), jnp.float32),
            pltpu.VMEM((P, FC), jnp.float32),
            pltpu.VMEM_SHARED((ACC_ROWS, FC), jnp.float32),
            pltpu.SemaphoreType.DMA,
            pltpu.SemaphoreType.DMA,
        ],
    )
    def k(src_h, dst_h, tab_h, z_h, out_h,
          sidx, didx, rbuf0, rbuf1, sbuf, acc, sem0, sem1):
        cid = lax.axis_index("c")
        sid = lax.axis_index("s")
        src_t = src_h.at[sid]
        dst_t = dst_h.at[sid]
        bufs = (rbuf0, rbuf1)
        sems = (sem0, sem1)

        def chunk(c_tab, c_out, g_lo, g_hi):
            pltpu.sync_copy(z_h, sbuf)
            _zero_acc(sid, sbuf, acc)
            plsc.subcore_barrier()
            tab_c = tab_h.at[c_tab]

            def group(g, carry):
                pltpu.sync_copy(src_t.at[pl.ds(g * G, G)], sidx)
                pltpu.sync_copy(dst_t.at[pl.ds(g * G, G)], didx)
                handles = [None, None]
                handles[0] = pltpu.async_copy(
                    tab_c.at[sidx.at[0]], bufs[0], sems[0])
                for r in range(G):
                    if r + 1 < G:
                        handles[(r + 1) % 2] = pltpu.async_copy(
                            tab_c.at[sidx.at[r + 1]],
                            bufs[(r + 1) % 2], sems[(r + 1) % 2])
                    handles[r % 2].wait()
                    pltpu.sync_copy(bufs[r % 2], acc.at[didx.at[r]], add=True)
                return carry

            lax.fori_loop(g_lo, g_hi, group, 0)
            plsc.subcore_barrier()
            _writeback(sid, sbuf, acc, out_h.at[c_out])

        for l in range(nfull):
            if l > 0:
                plsc.subcore_barrier()
            chunk(cid + 2 * l, cid + 2 * l, 0, NGRP)
        if split_last:
            if nfull > 0:
                plsc.subcore_barrier()
            half = NGRP // 2
            chunk(nchunk - 1, nchunk - 1 + cid,
                  cid * half, cid * half + half)
        elif nchunk % 2 == 1:
            if nfull > 0:
                plsc.subcore_barrier()

            @pl.when(cid == 0)
            def _():
                chunk(nchunk - 1, nchunk - 1, 0, NGRP)

    return k(src3, dst3, table, zeros_hbm)


def _counts_a(col3, pinb_hbm, zeros_hbm):
    @functools.partial(
        pl.kernel,
        mesh=_sc_mesh(),
        out_type=jax.ShapeDtypeStruct((2, NHE, 128), jnp.float32),
        scratch_types=[
            pltpu.VMEM((NBLK_C, KB), jnp.int32),
            pltpu.VMEM((KB, 128), jnp.float32),
            pltpu.VMEM((P, 128), jnp.float32),
            pltpu.VMEM_SHARED((ACC_ROWS, 128), jnp.float32),
            pltpu.SemaphoreType.DMA,
        ],
    )
    def k(col_h, pin_h, z_h, qb_out, colv, pbuf, sbuf, acc, sem):
        cid = lax.axis_index("c")
        sid = lax.axis_index("s")
        j = cid * 16 + sid
        pltpu.sync_copy(col_h.at[j], colv)
        pltpu.sync_copy(z_h, sbuf)
        _zero_acc(sid, sbuf, acc)
        plsc.subcore_barrier()
        ebase = j * (NBLK_C * KB)

        def body(b, carry):
            pltpu.async_copy(pin_h.at[pl.ds(ebase + b * KB, KB)], pbuf,
                             sem).wait()
            pltpu.sync_copy(pbuf, acc.at[colv.at[b]], add=True)
            return carry

        lax.fori_loop(0, NBLK_C, body, 0)
        plsc.subcore_barrier()
        _writeback(sid, sbuf, acc, qb_out.at[cid])

    return k(col3, pinb_hbm, zeros_hbm)


def _counts_b(row3, macro2, ones0, ones1, zeros_hbm):
    @functools.partial(
        pl.kernel,
        mesh=_sc_mesh(),
        out_type=jax.ShapeDtypeStruct((2, N, 128), jnp.float32),
        scratch_types=[
            pltpu.VMEM((NBLK_C, KB), jnp.int32),
            pltpu.VMEM((KB, 128), jnp.float32),
            pltpu.VMEM((KB, 128), jnp.float32),
            pltpu.VMEM((4, KB), jnp.int32),
            pltpu.VMEM((P, 128), jnp.float32),
            pltpu.VMEM_SHARED((ACC_ROWS, 128), jnp.float32),
        ],
    )
    def k(row_h, mac_h, o0_h, o1_h, z_h, dm_out,
          rowv, o0b, o1b, macv, sbuf, acc):
        cid = lax.axis_index("c")
        sid = lax.axis_index("s")
        j = cid * 16 + sid
        pltpu.sync_copy(row_h.at[j], rowv)
        pltpu.sync_copy(o0_h, o0b)
        pltpu.sync_copy(o1_h, o1b)
        pltpu.sync_copy(mac_h, macv)
        pltpu.sync_copy(z_h, sbuf)
        _zero_acc(sid, sbuf, acc)
        plsc.subcore_barrier()

        def body(b, carry):
            pltpu.sync_copy(o0b, acc.at[rowv.at[b]], add=True)
            return carry

        lax.fori_loop(0, NBLK_C, body, 0)

        @pl.when(sid < 2)
        def _():
            pltpu.sync_copy(o1b, acc.at[macv.at[cid * 2 + sid]], add=True)

        plsc.subcore_barrier()
        _writeback(sid, sbuf, acc, dm_out.at[cid])

    return k(row3, macro2, ones0, ones1, zeros_hbm)


BN = 2000
BNM = 2000


def _prep(x, fake_pos, dm):
    def body(x_ref, fp_ref, dm_ref, o_ref):
        xb = x_ref[...]
        m = dm_ref[0, :, 1:2] + dm_ref[1, :, 1:2]
        ism = (m > 0).astype(jnp.float32)
        z = jnp.zeros((BN, 125), jnp.float32)
        o_ref[0] = xb[:, :128]
        o_ref[1] = xb[:, 128:]
        o_ref[2] = jnp.concatenate([fp_ref[...], ism, z], axis=1)

    return pl.pallas_call(
        body,
        grid=(N // BN,),
        in_specs=[
            pl.BlockSpec((BN, FIN), lambda r: (r, 0)),
            pl.BlockSpec((BN, 2), lambda r: (r, 0)),
            pl.BlockSpec((2, BN, 128), lambda r: (0, r, 0)),
        ],
        out_specs=pl.BlockSpec((3, BN, 128), lambda r: (0, r, 0)),
        out_shape=jax.ShapeDtypeStruct((3, N, 128), jnp.float32),
    )(x, fake_pos, dm)


def _scale1(u, qb):
    def body(u_ref, qb_ref, o_ref):
        bsum = qb_ref[0, :, 16:17] + qb_ref[1, :, 16:17]
        binv = jnp.where(bsum > 0, 1.0 / bsum, 0.0)
        q = qb_ref[0, :, :16] + qb_ref[1, :, :16]
        c2 = u_ref[2] + u_ref[3]
        o_ref[0] = u_ref[0] * binv
        o_ref[1] = u_ref[1] * binv
        o_ref[2] = jnp.concatenate(
            [c2[:, :3], q, jnp.zeros((BN, 109), jnp.float32)],
            axis=1) * binv

    return pl.pallas_call(
        body,
        grid=(NHE // BN,),
        in_specs=[
            pl.BlockSpec((4, BN, 128), lambda r: (0, r, 0)),
            pl.BlockSpec((2, BN, 128), lambda r: (0, r, 0)),
        ],
        out_specs=pl.BlockSpec((3, BN, 128), lambda r: (0, r, 0)),
        out_shape=jax.ShapeDtypeStruct((3, NHE, 128), jnp.float32),
    )(u, qb)


def _scale(u, qb):
    def body(u_ref, qb_ref, o_ref):
        bsum = qb_ref[0, :, 16:17] + qb_ref[1, :, 16:17]
        binv = jnp.where(bsum > 0, 1.0 / bsum, 0.0)
        o_ref[...] = u_ref[...] * binv[None]

    return pl.pallas_call(
        body,
        grid=(NHE // BN,),
        in_specs=[
            pl.BlockSpec((4, BN, 128), lambda r: (0, r, 0)),
            pl.BlockSpec((2, BN, 128), lambda r: (0, r, 0)),
        ],
        out_specs=pl.BlockSpec((4, BN, 128), lambda r: (0, r, 0)),
        out_shape=jax.ShapeDtypeStruct((4, NHE, 128), jnp.float32),
    )(u, qb)


def _layer_matmul(v, wc, b2d, dm, nc):
    def body(v_ref, w_ref, b_ref, dm_ref, o_ref):
        acc = jnp.zeros((BNM, 128), jnp.float32)
        for kk in range(nc):
            acc = acc + jnp.dot(v_ref[kk], w_ref[kk],
                                preferred_element_type=jnp.float32)
        dsum = dm_ref[0, :, 0:1] + dm_ref[1, :, 0:1]
        dinv = jnp.where(dsum > 0, 1.0 / dsum, 0.0)
        o_ref[0] = _leaky(acc * dinv + b_ref[0, 0][None])

    return pl.pallas_call(
        body,
        grid=(N // BNM, 4),
        in_specs=[
            pl.BlockSpec((nc, BNM, 128), lambda r, c: (0, r, 0)),
            pl.BlockSpec((nc, 128, 128), lambda r, c: (0, 0, c)),
            pl.BlockSpec((1, 1, 128), lambda r, c: (c, 0, 0)),
            pl.BlockSpec((2, BNM, 128), lambda r, c: (0, r, 0)),
        ],
        out_specs=pl.BlockSpec((1, BNM, 128), lambda r, c: (c, r, 0)),
        out_shape=jax.ShapeDtypeStruct((4, N, 128), jnp.float32),
    )(v, wc, b2d, dm)


def _pool(hc, batch2d, dm):
    RN = N // BN

    def body(h_ref, b_ref, dm_ref, o_ref, s_ref, sm_ref, c_ref, cm_ref):
        r = pl.program_id(0)

        @pl.when(r == 0)
        def _():
            s_ref[...] = jnp.zeros_like(s_ref)
            sm_ref[...] = jnp.zeros_like(sm_ref)
            c_ref[...] = jnp.zeros_like(c_ref)
            cm_ref[...] = jnp.zeros_like(cm_ref)

        ids = b_ref[...]
        oh = (ids == lax.broadcasted_iota(jnp.int32, (BN, NGRAPH), 1))
        oh = oh.astype(jnp.float32)
        mcnt = dm_ref[0, :, 1:2] + dm_ref[1, :, 1:2]
        h = jnp.concatenate(
            [h_ref[0], h_ref[1], h_ref[2], h_ref[3]], axis=1)
        dn = (((0,), (0,)), ((), ()))
        s_ref[...] += lax.dot_general(oh, h, dn,
                                      preferred_element_type=jnp.float32)
        sm_ref[...] += lax.dot_general(oh, mcnt * h, dn,
                                       preferred_element_type=jnp.float32)
        ones = jnp.ones((BN, 1), jnp.float32)
        c_ref[...] += lax.dot_general(oh, ones, dn,
                                      preferred_element_type=jnp.float32)
        cm_ref[...] += lax.dot_general(oh, mcnt, dn,
                                       preferred_element_type=jnp.float32)

        @pl.when(r == RN - 1)
        def _():
            sm = sm_ref[...] / jnp.maximum(cm_ref[...], 1.0)
            s = s_ref[...] / jnp.maximum(c_ref[...], 1.0)
            o_ref[...] = jnp.concatenate([sm, s], axis=1)

    return pl.pallas_call(
        body,
        grid=(RN,),
        in_specs=[
            pl.BlockSpec((4, BN, 128), lambda r: (0, r, 0)),
            pl.BlockSpec((BN, 1), lambda r: (r, 0)),
            pl.BlockSpec((2, BN, 128), lambda r: (0, r, 0)),
        ],
        out_specs=pl.BlockSpec((NGRAPH, 2 * NHID), lambda r: (0, 0)),
        out_shape=jax.ShapeDtypeStruct((NGRAPH, 2 * NHID), jnp.float32),
        scratch_shapes=[
            pltpu.VMEM((NGRAPH, NHID), jnp.float32),
            pltpu.VMEM((NGRAPH, NHID), jnp.float32),
            pltpu.VMEM((NGRAPH, 1), jnp.float32),
            pltpu.VMEM((NGRAPH, 1), jnp.float32),
        ],
    )(hc, batch2d, dm)


def _mlp(x1, x2, x3, mw1, mb1, mw2, mb2, mw3, mb3):
    def body(x1_r, x2_r, x3_r, w1_r, b1_r, w2_r, b2_r, w3_r, b3_r, o_ref):
        g = x1_r[...] + x2_r[...] + x3_r[...]
        g = _leaky(jnp.dot(g, w1_r[...],
                           preferred_element_type=jnp.float32) + b1_r[...])
        g = _leaky(jnp.dot(g, w2_r[...],
                           preferred_element_type=jnp.float32) + b2_r[...])
        o_ref[...] = jnp.dot(g, w3_r[...],
                             preferred_element_type=jnp.float32) + b3_r[...]

    return pl.pallas_call(
        body,
        out_shape=jax.ShapeDtypeStruct((NGRAPH, 4), jnp.float32),
    )(x1, x2, x3, mw1, mb1.reshape(1, -1), mw2, mb2.reshape(1, -1),
      mw3, mb3.reshape(1, -1))


def kernel(x, edge_index, pin_feature, batch, fake_pos, macro_index,
           W1, pinW1, b1, W2, b2, W3, b3, mw1, mb1, mw2, mb2, mw3, mb3):
    row = edge_index[0].astype(jnp.int32)
    col = edge_index[1].astype(jnp.int32)
    pad = EPAD - E

    def pad_idx(a, fill):
        return jnp.concatenate([a, jnp.full((pad,), fill, jnp.int32)])

    row_s = pad_idx(row, 0).reshape(16, NBLK, KB)
    row_d = pad_idx(row, DUMMY).reshape(16, NBLK, KB)
    col_s = pad_idx(col, 0).reshape(16, NBLK, KB)
    col_d = pad_idx(col, DUMMY).reshape(16, NBLK, KB)
    row_c = pad_idx(row, DUMMY).reshape(32, NBLK_C, KB)
    col_c = pad_idx(col, DUMMY).reshape(32, NBLK_C, KB)
    pinb = jnp.concatenate(
        [pin_feature, jnp.ones((E, 1), jnp.float32),
         jnp.zeros((E, 111), jnp.float32)], axis=1)
    pinb = jnp.concatenate([pinb, jnp.zeros((pad, 128), jnp.float32)])
    macro2 = macro_index.astype(jnp.int32).reshape(4, KB)

    ones0 = jnp.zeros((KB, 128), jnp.float32).at[:, 0].set(1.0)
    ones1 = jnp.zeros((KB, 128), jnp.float32).at[:, 1].set(1.0)
    z128 = jnp.zeros((P, 128), jnp.float32)

    qb = _counts_a(col_c, pinb, z128)
    dm = _counts_b(row_c, macro2, ones0, ones1, z128)

    h0c = _prep(x, fake_pos, dm)
    u1 = _spmm(3, row_s, col_d, h0c, z128, split_last=True)
    u1s = _scale1(u1, qb)
    v1 = _spmm(3, col_s, row_d, u1s, z128, split_last=True)
    w_c2 = jnp.concatenate([
        W1[256:259], pinW1, jnp.zeros((109, NHID), jnp.float32)])
    bigW1 = jnp.concatenate([W1[:256], w_c2, w_c2]).reshape(4, 128, NHID)
    h1 = _layer_matmul(v1, bigW1, b1.reshape(4, 1, 128), dm, 4)

    w2c = W2.reshape(4, 128, NHID)
    u2 = _spmm(4, row_s, col_d, h1, z128)
    v2 = _spmm(4, col_s, row_d, _scale(u2, qb), z128)
    h2 = _layer_matmul(v2, w2c, b2.reshape(4, 1, 128), dm, 4)

    w3c = W3.reshape(4, 128, NHID)
    u3 = _spmm(4, row_s, col_d, h2, z128)
    v3 = _spmm(4, col_s, row_d, _scale(u3, qb), z128)
    h3 = _layer_matmul(v3, w3c, b3.reshape(4, 1, 128), dm, 4)

    batch2d = batch.astype(jnp.int32).reshape(N, 1)
    x1 = _pool(h1, batch2d, dm)
    x2 = _pool(h2, batch2d, dm)
    x3 = _pool(h3, batch2d, dm)

    return _mlp(x1, x2, x3, mw1, mb1, mw2, mb2, mw3, mb3)
